# Initial kernel scaffold; baseline (speedup 1.0000x reference)
import jax
import jax.numpy as jnp
from jax.experimental import pallas as pl


def kernel(x, table):
    raise NotImplementedError("write your pallas kernel here")


# speedup vs baseline: 3.2341x; 3.2341x over previous
import functools

import jax
import jax.numpy as jnp
from jax import lax
from jax.experimental import pallas as pl
from jax.experimental.pallas import tpu as pltpu
from jax.experimental.pallas import tpu_sc as plsc

NC, NS = 2, 16
NW = NC * NS
D = 64
CHUNK = 128
BLK = 512
CPB = BLK // CHUNK


def _emb_body(bpw, x_hbm, table_hbm, out_hbm,
              idx_v, rows_v, isem0, isem1, gsem, osem0, osem1):
    wid = lax.axis_index("s") * NC + lax.axis_index("c")
    isems = (isem0, isem1)
    osems = (osem0, osem1)

    def idx_copy(g, buf):
        return pltpu.make_async_copy(
            x_hbm.at[wid, g], idx_v.at[buf], isems[buf])

    def out_copy(g, buf):
        return pltpu.make_async_copy(
            rows_v.at[buf],
            out_hbm.at[pl.ds((wid * bpw + g) * BLK, BLK)],
            osems[buf])

    idx_copy(0, 0).start()
    idx_copy(1, 1).start()

    def body(i, _):
        for buf in range(2):
            g = i * 2 + buf
            idx_copy(g, buf).wait()
            pl.when(i > 0)(lambda: out_copy(g - 2, buf).wait())
            gathers = []
            for j in range(CPB):
                gathers.append(pltpu.make_async_copy(
                    table_hbm.at[idx_v.at[buf, j]],
                    rows_v.at[buf, pl.ds(j * CHUNK, CHUNK)],
                    gsem))
                gathers[-1].start()
            for h in gathers:
                h.wait()
            pl.when(g + 2 < bpw)(lambda: idx_copy(g + 2, buf).start())
            out_copy(g, buf).start()
        return _

    lax.fori_loop(0, bpw // 2, body, None)
    out_copy(bpw - 2, 0).wait()
    out_copy(bpw - 1, 1).wait()


def kernel(x, table):
    orig_shape = x.shape
    b = x.size
    assert b % (NW * BLK * 2) == 0
    bpw = b // (NW * BLK)
    x4 = x.reshape(NW, bpw, CPB, CHUNK).astype(jnp.int32)

    mesh = plsc.VectorSubcoreMesh(core_axis_name="c", subcore_axis_name="s")
    run = pl.kernel(
        functools.partial(_emb_body, bpw),
        out_type=jax.ShapeDtypeStruct((b, D), jnp.float32),
        mesh=mesh,
        scratch_types=[
            pltpu.VMEM((2, CPB, CHUNK), jnp.int32),
            pltpu.VMEM((2, BLK, D), jnp.float32),
            pltpu.SemaphoreType.DMA,
            pltpu.SemaphoreType.DMA,
            pltpu.SemaphoreType.DMA,
            pltpu.SemaphoreType.DMA,
            pltpu.SemaphoreType.DMA,
        ],
        compiler_params=pltpu.CompilerParams(use_tc_tiling_on_sc=False),
    )
    out = run(x4, table)
    return out.reshape(*orig_shape, D)


# speedup vs baseline: 5.8055x; 1.7951x over previous
import functools

import jax
import jax.numpy as jnp
from jax import lax
from jax.experimental import pallas as pl
from jax.experimental.pallas import tpu as pltpu
from jax.experimental.pallas import tpu_sc as plsc

NC, NS = 2, 16
NW = NC * NS
D = 64
CHUNK = 128
BLK = 512
CPB = BLK // CHUNK


def _emb_body(bpw, x_hbm, table_hbm, out_hbm,
              tbl_sp, idx_v, rows_v, isem0, isem1, gsem, osem0, osem1):
    wid = lax.axis_index("s") * NC + lax.axis_index("c")
    isems = (isem0, isem1)
    osems = (osem0, osem1)

    pl.when(lax.axis_index("s") == 0)(
        lambda: pltpu.sync_copy(table_hbm, tbl_sp))
    plsc.subcore_barrier()

    def idx_copy(g, buf):
        return pltpu.make_async_copy(
            x_hbm.at[wid, g], idx_v.at[buf], isems[buf])

    def out_copy(g, buf):
        return pltpu.make_async_copy(
            rows_v.at[buf],
            out_hbm.at[pl.ds((wid * bpw + g) * BLK, BLK)],
            osems[buf])

    idx_copy(0, 0).start()
    idx_copy(1, 1).start()

    def body(i, _):
        for buf in range(2):
            g = i * 2 + buf
            idx_copy(g, buf).wait()
            pl.when(i > 0)(lambda: out_copy(g - 2, buf).wait())
            gathers = []
            for j in range(CPB):
                gathers.append(pltpu.make_async_copy(
                    tbl_sp.at[idx_v.at[buf, j]],
                    rows_v.at[buf, pl.ds(j * CHUNK, CHUNK)],
                    gsem))
                gathers[-1].start()
            for h in gathers:
                h.wait()
            pl.when(g + 2 < bpw)(lambda: idx_copy(g + 2, buf).start())
            out_copy(g, buf).start()
        return _

    lax.fori_loop(0, bpw // 2, body, None)
    out_copy(bpw - 2, 0).wait()
    out_copy(bpw - 1, 1).wait()


def kernel(x, table):
    orig_shape = x.shape
    b = x.size
    assert b % (NW * BLK * 2) == 0
    bpw = b // (NW * BLK)
    x4 = x.reshape(NW, bpw, CPB, CHUNK).astype(jnp.int32)

    mesh = plsc.VectorSubcoreMesh(core_axis_name="c", subcore_axis_name="s")
    run = pl.kernel(
        functools.partial(_emb_body, bpw),
        out_type=jax.ShapeDtypeStruct((b, D), jnp.float32),
        mesh=mesh,
        scratch_types=[
            pltpu.VMEM_SHARED((231, D), jnp.float32),
            pltpu.VMEM((2, CPB, CHUNK), jnp.int32),
            pltpu.VMEM((2, BLK, D), jnp.float32),
            pltpu.SemaphoreType.DMA,
            pltpu.SemaphoreType.DMA,
            pltpu.SemaphoreType.DMA,
            pltpu.SemaphoreType.DMA,
            pltpu.SemaphoreType.DMA,
        ],
        compiler_params=pltpu.CompilerParams(use_tc_tiling_on_sc=False),
    )
    out = run(x4, table)
    return out.reshape(*orig_shape, D)


# speedup vs baseline: 5.8296x; 1.0042x over previous
import functools

import jax
import jax.numpy as jnp
from jax import lax
from jax.experimental import pallas as pl
from jax.experimental.pallas import tpu as pltpu
from jax.experimental.pallas import tpu_sc as plsc

NC, NS = 2, 16
NW = NC * NS
D = 64
CHUNK = 512
BLK = 512
CPB = BLK // CHUNK


def _emb_body(bpw, x_hbm, table_hbm, out_hbm,
              tbl_sp, idx_v, rows_v, isem0, isem1, gsem, osem0, osem1):
    wid = lax.axis_index("s") * NC + lax.axis_index("c")
    isems = (isem0, isem1)
    osems = (osem0, osem1)

    pl.when(lax.axis_index("s") == 0)(
        lambda: pltpu.sync_copy(table_hbm, tbl_sp))
    plsc.subcore_barrier()

    def idx_copy(g, buf):
        return pltpu.make_async_copy(
            x_hbm.at[wid, g], idx_v.at[buf], isems[buf])

    def out_copy(g, buf):
        return pltpu.make_async_copy(
            rows_v.at[buf],
            out_hbm.at[pl.ds((wid * bpw + g) * BLK, BLK)],
            osems[buf])

    idx_copy(0, 0).start()
    idx_copy(1, 1).start()

    def body(i, _):
        for buf in range(2):
            g = i * 2 + buf
            idx_copy(g, buf).wait()
            pl.when(i > 0)(lambda: out_copy(g - 2, buf).wait())
            gathers = []
            for j in range(CPB):
                gathers.append(pltpu.make_async_copy(
                    tbl_sp.at[idx_v.at[buf, j]],
                    rows_v.at[buf, pl.ds(j * CHUNK, CHUNK)],
                    gsem))
                gathers[-1].start()
            for h in gathers:
                h.wait()
            pl.when(g + 2 < bpw)(lambda: idx_copy(g + 2, buf).start())
            out_copy(g, buf).start()
        return _

    lax.fori_loop(0, bpw // 2, body, None)
    out_copy(bpw - 2, 0).wait()
    out_copy(bpw - 1, 1).wait()


def kernel(x, table):
    orig_shape = x.shape
    b = x.size
    assert b % (NW * BLK * 2) == 0
    bpw = b // (NW * BLK)
    x4 = x.reshape(NW, bpw, CPB, CHUNK).astype(jnp.int32)

    mesh = plsc.VectorSubcoreMesh(core_axis_name="c", subcore_axis_name="s")
    run = pl.kernel(
        functools.partial(_emb_body, bpw),
        out_type=jax.ShapeDtypeStruct((b, D), jnp.float32),
        mesh=mesh,
        scratch_types=[
            pltpu.VMEM_SHARED((231, D), jnp.float32),
            pltpu.VMEM((2, CPB, CHUNK), jnp.int32),
            pltpu.VMEM((2, BLK, D), jnp.float32),
            pltpu.SemaphoreType.DMA,
            pltpu.SemaphoreType.DMA,
            pltpu.SemaphoreType.DMA,
            pltpu.SemaphoreType.DMA,
            pltpu.SemaphoreType.DMA,
        ],
        compiler_params=pltpu.CompilerParams(use_tc_tiling_on_sc=False),
    )
    out = run(x4, table)
    return out.reshape(*orig_shape, D)


# speedup vs baseline: 7.4720x; 1.2817x over previous
import functools

import jax
import jax.numpy as jnp
from jax import lax
from jax.experimental import pallas as pl
from jax.experimental.pallas import tpu as pltpu
from jax.experimental.pallas import tpu_sc as plsc

NC, NS = 2, 16
NW = NC * NS
D = 64
V = 231
VP = 232
NB = 16384
NJ = 200
SPAN = NB // NW
NV = SPAN // 16
TBL = D * VP


def _emb_body(xT_hbm, tbl_hbm, out_hbm,
              tbl_v, idx_v, out_buf, isem0, isem1, osem0, osem1):
    wid = lax.axis_index("s") * NC + lax.axis_index("c")
    base = wid * SPAN
    isems = (isem0, isem1)
    osems = (osem0, osem1)

    pltpu.sync_copy(tbl_hbm, tbl_v)

    def idx_copy(j, buf):
        return pltpu.make_async_copy(
            xT_hbm.at[j, pl.ds(base, SPAN)], idx_v.at[buf], isems[buf])

    def out_copy(j, buf):
        return pltpu.make_async_copy(
            out_buf.at[buf],
            out_hbm.at[j, :, pl.ds(wid * (SPAN // 128), SPAN // 128)],
            osems[buf])

    idx_copy(0, 0).start()
    idx_copy(1, 1).start()

    def col(i, buf):
        j = i * 2 + buf
        idx_copy(j, buf).wait()
        pl.when(i > 0)(lambda: out_copy(j - 2, buf).wait())

        def gathers(v, carry):
            xq = idx_v[buf, pl.ds(v * 16, 16)]
            i0 = v // 8
            b0 = (v % 8) * 16
            for k in range(D):
                idx = xq + k * VP
                out_buf[buf, k // 8, i0, k % 8, pl.ds(b0, 16)] = (
                    plsc.load_gather(tbl_v, [idx]))
            return carry

        lax.fori_loop(0, NV, gathers, 0)
        out_copy(j, buf).start()
        pl.when(j + 2 < NJ)(lambda: idx_copy(j + 2, buf).start())

    def body(i, carry):
        col(i, 0)
        col(i, 1)
        return carry

    lax.fori_loop(0, NJ // 2, body, 0)
    out_copy(NJ - 2, 0).wait()
    out_copy(NJ - 1, 1).wait()


def kernel(x, table):
    xT = x.T.astype(jnp.int32)
    tbl = jnp.pad(table.T, ((0, 0), (0, VP - V))).reshape(-1)

    mesh = plsc.VectorSubcoreMesh(core_axis_name="c", subcore_axis_name="s")
    run = pl.kernel(
        _emb_body,
        out_type=jax.ShapeDtypeStruct((NJ, D // 8, NB // 128, 8, 128),
                                      jnp.float32),
        mesh=mesh,
        scratch_types=[
            pltpu.VMEM((TBL,), jnp.float32),
            pltpu.VMEM((2, SPAN), jnp.int32),
            pltpu.VMEM((2, D // 8, SPAN // 128, 8, 128), jnp.float32),
            pltpu.SemaphoreType.DMA,
            pltpu.SemaphoreType.DMA,
            pltpu.SemaphoreType.DMA,
            pltpu.SemaphoreType.DMA,
        ],
        compiler_params=pltpu.CompilerParams(use_tc_tiling_on_sc=False,
                                             needs_layout_passes=False),
    )
    out5 = run(xT, tbl)
    return jnp.transpose(out5, (2, 4, 0, 1, 3)).reshape(NB, NJ, D)


# speedup vs baseline: 15.0242x; 2.0107x over previous
import functools

import jax
import jax.numpy as jnp
from jax import lax
from jax.experimental import pallas as pl
from jax.experimental.pallas import tpu as pltpu
from jax.experimental.pallas import tpu_sc as plsc

NC, NS = 2, 16
NW = NC * NS
D = 64
V = 231
VP = 232
NB = 16384
NJ = 200
SPAN = NB // NW
NV = SPAN // 16
TBL = D * VP


def _emb_body(xT_hbm, tbl_hbm, out_hbm,
              tbl_v, idx_v, out_buf, isem0, isem1, osem0, osem1):
    wid = lax.axis_index("s") * NC + lax.axis_index("c")
    base = wid * SPAN
    isems = (isem0, isem1)
    osems = (osem0, osem1)

    pltpu.sync_copy(tbl_hbm, tbl_v)

    def idx_copy(j, buf):
        return pltpu.make_async_copy(
            xT_hbm.at[j, pl.ds(base, SPAN)], idx_v.at[buf], isems[buf])

    def out_copy(j, buf):
        return pltpu.make_async_copy(
            out_buf.at[buf],
            out_hbm.at[j, :, pl.ds(wid * (SPAN // 128), SPAN // 128)],
            osems[buf])

    idx_copy(0, 0).start()
    idx_copy(1, 1).start()

    def col(i, buf):
        j = i * 2 + buf
        idx_copy(j, buf).wait()
        pl.when(i > 0)(lambda: out_copy(j - 2, buf).wait())

        @plsc.parallel_loop(0, NV, unroll=2)
        def gathers(v):
            xq = idx_v[buf, pl.ds(v * 16, 16)]
            i0 = v // 8
            b0 = (v % 8) * 16
            for k in range(D):
                idx = xq + k * VP
                out_buf[buf, k // 8, i0, k % 8, pl.ds(b0, 16)] = (
                    plsc.load_gather(tbl_v, [idx]))
        out_copy(j, buf).start()
        pl.when(j + 2 < NJ)(lambda: idx_copy(j + 2, buf).start())

    def body(i, carry):
        col(i, 0)
        col(i, 1)
        return carry

    lax.fori_loop(0, NJ // 2, body, 0)
    out_copy(NJ - 2, 0).wait()
    out_copy(NJ - 1, 1).wait()


def kernel(x, table):
    xT = x.T.astype(jnp.int32)
    tbl = jnp.pad(table.T, ((0, 0), (0, VP - V))).reshape(-1)

    mesh = plsc.VectorSubcoreMesh(core_axis_name="c", subcore_axis_name="s")
    run = pl.kernel(
        _emb_body,
        out_type=jax.ShapeDtypeStruct((NJ, D // 8, NB // 128, 8, 128),
                                      jnp.float32),
        mesh=mesh,
        scratch_types=[
            pltpu.VMEM((TBL,), jnp.float32),
            pltpu.VMEM((2, SPAN), jnp.int32),
            pltpu.VMEM((2, D // 8, SPAN // 128, 8, 128), jnp.float32),
            pltpu.SemaphoreType.DMA,
            pltpu.SemaphoreType.DMA,
            pltpu.SemaphoreType.DMA,
            pltpu.SemaphoreType.DMA,
        ],
        compiler_params=pltpu.CompilerParams(use_tc_tiling_on_sc=False,
                                             needs_layout_passes=False),
    )
    out5 = run(xT, tbl)
    return jnp.transpose(out5, (2, 4, 0, 1, 3)).reshape(NB, NJ, D)


# speedup vs baseline: 22.3457x; 1.4873x over previous
import functools

import jax
import jax.numpy as jnp
from jax import lax
from jax.experimental import pallas as pl
from jax.experimental.pallas import tpu as pltpu
from jax.experimental.pallas import tpu_sc as plsc

NC, NS = 2, 16
NW = NC * NS
D = 64
V = 231
VP = 232
NB = 16384
NJ = 200
SPAN = NB // NW
NV = SPAN // 16
TBL = D * VP


def _emb_body(xT_hbm, tbl_hbm, out_hbm,
              tbl_v, idx_v, out_buf, isem0, isem1, osem0, osem1):
    wid = lax.axis_index("s") * NC + lax.axis_index("c")
    base = wid * SPAN
    isems = (isem0, isem1)
    osems = (osem0, osem1)

    pltpu.sync_copy(tbl_hbm, tbl_v)

    def idx_copy(j, buf):
        return pltpu.make_async_copy(
            xT_hbm.at[j, pl.ds(base, SPAN)], idx_v.at[buf], isems[buf])

    def out_copy(j, buf):
        return pltpu.make_async_copy(
            out_buf.at[buf],
            out_hbm.at[j, :, pl.ds(wid * (SPAN // 128), SPAN // 128)],
            osems[buf])

    idx_copy(0, 0).start()
    idx_copy(1, 1).start()

    def col(i, buf):
        j = i * 2 + buf
        idx_copy(j, buf).wait()
        pl.when(i > 0)(lambda: out_copy(j - 2, buf).wait())

        @plsc.parallel_loop(0, NV, unroll=4)
        def gathers(v):
            xq = idx_v[buf, pl.ds(v * 16, 16)]
            i0 = v // 8
            b0 = (v % 8) * 16
            for k in range(D):
                idx = xq + k * VP
                out_buf[buf, k // 8, i0, k % 8, pl.ds(b0, 16)] = (
                    plsc.load_gather(tbl_v, [idx]))
        out_copy(j, buf).start()
        pl.when(j + 2 < NJ)(lambda: idx_copy(j + 2, buf).start())

    def body(i, carry):
        col(i, 0)
        col(i, 1)
        return carry

    lax.fori_loop(0, NJ // 2, body, 0)
    out_copy(NJ - 2, 0).wait()
    out_copy(NJ - 1, 1).wait()


def kernel(x, table):
    xT = x.T.astype(jnp.int32)
    tbl = jnp.pad(table.T, ((0, 0), (0, VP - V))).reshape(-1)

    mesh = plsc.VectorSubcoreMesh(core_axis_name="c", subcore_axis_name="s")
    run = pl.kernel(
        _emb_body,
        out_type=jax.ShapeDtypeStruct((NJ, D // 8, NB // 128, 8, 128),
                                      jnp.float32),
        mesh=mesh,
        scratch_types=[
            pltpu.VMEM((TBL,), jnp.float32),
            pltpu.VMEM((2, SPAN), jnp.int32),
            pltpu.VMEM((2, D // 8, SPAN // 128, 8, 128), jnp.float32),
            pltpu.SemaphoreType.DMA,
            pltpu.SemaphoreType.DMA,
            pltpu.SemaphoreType.DMA,
            pltpu.SemaphoreType.DMA,
        ],
        compiler_params=pltpu.CompilerParams(use_tc_tiling_on_sc=False,
                                             needs_layout_passes=False),
    )
    out5 = run(xT, tbl)
    return jnp.transpose(out5, (2, 4, 0, 1, 3)).reshape(NB, NJ, D)
